# Initial kernel scaffold; baseline (speedup 1.0000x reference)
import jax
import jax.numpy as jnp
from jax.experimental import pallas as pl


def kernel(x, edge_index, W, att_src, att_dst, bias, Wl, bl):
    raise NotImplementedError("write your pallas kernel here")


# speedup vs baseline: 22.6795x; 22.6795x over previous
import functools

import jax
import jax.numpy as jnp
from jax import lax
from jax.experimental import pallas as pl
from jax.experimental.pallas import tpu as pltpu
from jax.experimental.pallas import tpu_sc as plsc

N = 10000
D = 128
E = 320000
E_TOT = E + N
NC = 2
NS = 16
NW = NC * NS
CHUNK = 128
NCHUNK = 81
EPW = CHUNK * NCHUNK
EP = EPW * NW
PAD = EP - E_TOT
ROWS_PT = 624
TAIL_OFF = ROWS_PT * NS
TAIL = N - TAIL_OFF
ZROWS = 104
NEG_SLOPE = 0.2


def _proj_body(x_ref, w_ref, att2_ref, h_ref, a2_ref):
    h = jnp.dot(x_ref[...], w_ref[...], preferred_element_type=jnp.float32)
    h_ref[...] = h
    a2_ref[...] = jnp.dot(h, att2_ref[...], preferred_element_type=jnp.float32)


def _project(x, W, att2):
    blk = 1000
    return pl.pallas_call(
        _proj_body,
        grid=(N // blk,),
        in_specs=[
            pl.BlockSpec((blk, D), lambda i: (i, 0)),
            pl.BlockSpec((D, D), lambda i: (0, 0)),
            pl.BlockSpec((D, 8), lambda i: (0, 0)),
        ],
        out_specs=[
            pl.BlockSpec((blk, D), lambda i: (i, 0)),
            pl.BlockSpec((blk, 8), lambda i: (i, 0)),
        ],
        out_shape=[
            jax.ShapeDtypeStruct((N, D), jnp.float32),
            jax.ShapeDtypeStruct((N, 8), jnp.float32),
        ],
    )(x, W, att2)


def _gat_sc_body(a_s_hbm, a_d_hbm, src_hbm, dst3_hbm, h_hbm,
                 num_hbm, denp_hbm,
                 dst_v, den_v, buf_v, src_c, asg_c, adg_c, ex_c,
                 sem, sem2, sem3, num_sh):
    cid = lax.axis_index("c")
    sid = lax.axis_index("s")
    wid = cid * NS + sid
    base = wid * EPW

    pltpu.sync_copy(dst3_hbm.at[wid], dst_v)

    zero16 = jnp.zeros((16,), jnp.float32)

    def _zrow(r, _):
        for j in range(D // 16):
            buf_v[r, pl.ds(j * 16, 16)] = zero16
        return 0
    lax.fori_loop(0, ZROWS, _zrow, 0)

    def _zcopy(k, _):
        pltpu.sync_copy(buf_v.at[pl.ds(0, ZROWS)],
                        num_sh.at[pl.ds(sid * ROWS_PT + k * ZROWS, ZROWS)])
        return 0
    lax.fori_loop(0, ROWS_PT // ZROWS, _zcopy, 0)

    @pl.when(sid == 0)
    def _ztail():
        pltpu.sync_copy(buf_v.at[pl.ds(0, TAIL)],
                        num_sh.at[pl.ds(TAIL_OFF, TAIL)])

    def _zden(i, _):
        den_v[pl.ds(i * 16, 16)] = zero16
        return 0
    lax.fori_loop(0, N // 16, _zden, 0)

    plsc.subcore_barrier()

    def _p(k, _):
        pltpu.sync_copy(src_hbm.at[pl.ds(base + k * CHUNK, CHUNK)], src_c)
        cp1 = pltpu.async_copy(a_s_hbm.at[src_c], asg_c, sem)
        cp2 = pltpu.async_copy(a_d_hbm.at[dst_v.at[k]], adg_c, sem2)
        cp3 = pltpu.async_copy(h_hbm.at[src_c], buf_v, sem3)
        cp1.wait()
        cp2.wait()

        for t in range(CHUNK // 16):
            e = asg_c[pl.ds(t * 16, 16)] + adg_c[pl.ds(t * 16, 16)]
            e = jnp.where(e > 0, e, NEG_SLOPE * e)
            gid = base + k * CHUNK + t * 16 + lax.iota(jnp.int32, 16)
            ex = jnp.where(gid < E_TOT, jnp.exp(e), 0.0)
            ex_c[pl.ds(t * 16, 16)] = ex
            didx = dst_v[k, pl.ds(t * 16, 16)]
            plsc.addupdate_scatter(den_v, [didx], ex)

        cp3.wait()

        def _scale(r, _):
            sv = plsc.load_gather(ex_c, [jnp.full((16,), r, jnp.int32)])
            for j in range(D // 16):
                buf_v[r, pl.ds(j * 16, 16)] = buf_v[r, pl.ds(j * 16, 16)] * sv
            return 0
        lax.fori_loop(0, CHUNK, _scale, 0)

        pltpu.sync_copy(buf_v, num_sh.at[dst_v.at[k]], add=True)
        return 0
    lax.fori_loop(0, NCHUNK, _p, 0)

    pltpu.sync_copy(den_v, denp_hbm.at[wid])

    plsc.subcore_barrier()
    pltpu.sync_copy(num_sh.at[pl.ds(sid * ROWS_PT, ROWS_PT)],
                    num_hbm.at[cid].at[pl.ds(sid * ROWS_PT, ROWS_PT)])

    @pl.when(sid == 0)
    def _ntail():
        pltpu.sync_copy(num_sh.at[pl.ds(TAIL_OFF, TAIL)],
                        num_hbm.at[cid].at[pl.ds(TAIL_OFF, TAIL)])


_gat_sc = functools.partial(
    pl.kernel,
    out_type=[
        jax.ShapeDtypeStruct((NC, N, D), jnp.float32),
        jax.ShapeDtypeStruct((NW, N), jnp.float32),
    ],
    mesh=plsc.VectorSubcoreMesh(core_axis_name="c", subcore_axis_name="s"),
    compiler_params=pltpu.CompilerParams(needs_layout_passes=False),
    scratch_types=[
        pltpu.VMEM((NCHUNK, CHUNK), jnp.int32),
        pltpu.VMEM((N,), jnp.float32),
        pltpu.VMEM((CHUNK, D), jnp.float32),
        pltpu.VMEM((CHUNK,), jnp.int32),
        pltpu.VMEM((CHUNK,), jnp.float32),
        pltpu.VMEM((CHUNK,), jnp.float32),
        pltpu.VMEM((CHUNK,), jnp.float32),
        pltpu.SemaphoreType.DMA,
        pltpu.SemaphoreType.DMA,
        pltpu.SemaphoreType.DMA,
        pltpu.VMEM_SHARED((N, D), jnp.float32),
    ],
)(_gat_sc_body)


def _fin_body(num_ref, den_ref, bias_ref, wl_ref, bl_ref, out_ref, acc_ref):
    i = pl.program_id(0)

    @pl.when(i == 0)
    def _():
        acc_ref[...] = jnp.zeros_like(acc_ref)

    nm = num_ref[0] + num_ref[1]
    dn = jnp.sum(den_ref[...], axis=1)
    o = jnp.maximum(nm / dn[:, None] + bias_ref[...][None, :], 0.0)
    acc_ref[...] += jnp.sum(o, axis=0, keepdims=True)

    @pl.when(i == pl.num_programs(0) - 1)
    def _():
        out_ref[...] = (
            jnp.dot(acc_ref[...] * (1.0 / N), wl_ref[...],
                    preferred_element_type=jnp.float32)
            + bl_ref[...][None, :]
        )


def _finalize(num, denp, bias, Wl, bl):
    blk = 1000
    return pl.pallas_call(
        _fin_body,
        grid=(N // blk,),
        in_specs=[
            pl.BlockSpec((NC, blk, D), lambda i: (0, i, 0)),
            pl.BlockSpec((blk, NW), lambda i: (i, 0)),
            pl.BlockSpec((D,), lambda i: (0,)),
            pl.BlockSpec((D, D), lambda i: (0, 0)),
            pl.BlockSpec((D,), lambda i: (0,)),
        ],
        out_specs=pl.BlockSpec((1, D), lambda i: (0, 0)),
        out_shape=jax.ShapeDtypeStruct((1, D), jnp.float32),
        scratch_shapes=[pltpu.VMEM((1, D), jnp.float32)],
    )(num, denp, bias, Wl, bl)


def kernel(x, edge_index, W, att_src, att_dst, bias, Wl, bl):
    loop = jnp.arange(N, dtype=jnp.int32)
    padz = jnp.zeros((PAD,), jnp.int32)
    src_p = jnp.concatenate([edge_index[0], loop, padz])
    dst_p = jnp.concatenate([edge_index[1], loop, padz])
    dst3 = dst_p.reshape(NW, NCHUNK, CHUNK)
    att2 = jnp.zeros((D, 8), jnp.float32)
    att2 = att2.at[:, 0].set(att_src).at[:, 1].set(att_dst)

    h, a2 = _project(x, W, att2)
    a_s = a2[:, 0]
    a_d = a2[:, 1]

    num, denp = _gat_sc(a_s, a_d, src_p, dst3, h)
    return _finalize(num, denp.T, bias, Wl, bl)
